# baseline (speedup 1.0000x reference)
import functools

import jax
import jax.numpy as jnp
from jax import lax
from jax.experimental import pallas as pl
from jax.experimental.pallas import tpu as pltpu
from jax.experimental.pallas import tpu_sc as plsc

_VOCAB = 1_000_000
_D = 64
_BATCH = 4096
_HIST = 200
_TOT = _BATCH * _HIST
_NC = 2
_NS = 16
_NW = _NC * _NS
_IW = 128
_K = 4
_CHUNK = _K * _IW
_ROWS_PER_W = _TOT // _NW
_N_CHUNK = _ROWS_PER_W // _CHUNK
_IDX_ROWS_PER_W = _ROWS_PER_W // _IW

_mesh = plsc.VectorSubcoreMesh(core_axis_name="c", subcore_axis_name="s")


@functools.partial(
    pl.kernel,
    mesh=_mesh,
    compiler_params=pltpu.CompilerParams(use_tc_tiling_on_sc=False),
    out_type=jax.ShapeDtypeStruct((_TOT, _D), jnp.float32),
    scratch_types=[
        pltpu.VMEM((_K, _IW), jnp.int32),
        pltpu.VMEM((_CHUNK, _D), jnp.float32),
        pltpu.SemaphoreType.DMA,
    ],
)
def _lookup(table_hbm, idx_hbm, out_hbm, idx_v, rows_v, sem):
    wid = lax.axis_index("s") * _NC + lax.axis_index("c")
    row0 = wid * _IDX_ROWS_PER_W

    def body(g, _):
        base = row0 + g * _K
        pltpu.sync_copy(idx_hbm.at[pl.ds(base, _K)], idx_v)
        copies = [
            pltpu.async_copy(
                table_hbm.at[idx_v.at[j]],
                rows_v.at[pl.ds(j * _IW, _IW)],
                sem,
            )
            for j in range(_K)
        ]
        for c in copies:
            c.wait()
        pltpu.sync_copy(rows_v, out_hbm.at[pl.ds(base * _IW, _CHUNK)])
        return 0

    lax.fori_loop(0, _N_CHUNK, body, 0)


def kernel(input_ids, embed_tokens_weight):
    idx2d = input_ids.reshape(_TOT // _IW, _IW)
    out = _lookup(embed_tokens_weight, idx2d)
    return out.reshape(_BATCH, _HIST, _D)


# speedup vs baseline: 1.0470x; 1.0470x over previous
import functools

import jax
import jax.numpy as jnp
from jax import lax
from jax.experimental import pallas as pl
from jax.experimental.pallas import tpu as pltpu
from jax.experimental.pallas import tpu_sc as plsc

_VOCAB = 1_000_000
_D = 64
_BATCH = 4096
_HIST = 200
_TOT = _BATCH * _HIST
_NC = 2
_NS = 16
_NW = _NC * _NS
_IW = 128
_K = 4
_CHUNK = _K * _IW
_ROWS_PER_W = _TOT // _NW
_N_CHUNK = _ROWS_PER_W // _CHUNK
_IDX_ROWS_PER_W = _ROWS_PER_W // _IW

_mesh = plsc.VectorSubcoreMesh(core_axis_name="c", subcore_axis_name="s")


@functools.partial(
    pl.kernel,
    mesh=_mesh,
    compiler_params=pltpu.CompilerParams(use_tc_tiling_on_sc=False),
    out_type=jax.ShapeDtypeStruct((_TOT, _D), jnp.float32),
    scratch_types=[
        pltpu.VMEM((_K, _IW), jnp.int32),
        pltpu.VMEM((_K, _IW), jnp.int32),
        pltpu.VMEM((_CHUNK, _D), jnp.float32),
        pltpu.VMEM((_CHUNK, _D), jnp.float32),
        pltpu.SemaphoreType.DMA,
        pltpu.SemaphoreType.DMA,
        pltpu.SemaphoreType.DMA,
        pltpu.SemaphoreType.DMA,
    ],
)
def _lookup(table_hbm, idx_hbm, out_hbm, idx0, idx1, rows0, rows1,
            g0, g1, o0, o1):
    wid = lax.axis_index("s") * _NC + lax.axis_index("c")
    row0 = wid * _IDX_ROWS_PER_W
    idx_v = (idx0, idx1)
    rows_v = (rows0, rows1)
    gsem = (g0, g1)
    osem = (o0, o1)

    def fire_gathers(b):
        for j in range(_K):
            pltpu.async_copy(
                table_hbm.at[idx_v[b].at[j]],
                rows_v[b].at[pl.ds(j * _IW, _IW)],
                gsem[b],
            )

    def wait_gathers(b):
        for j in range(_K):
            pltpu.make_async_copy(
                table_hbm.at[idx_v[b].at[j]],
                rows_v[b].at[pl.ds(j * _IW, _IW)],
                gsem[b],
            ).wait()

    for b in range(2):
        pltpu.sync_copy(idx_hbm.at[pl.ds(row0 + b * _K, _K)], idx_v[b])
        fire_gathers(b)

    def pair_body(i, carry):
        for b in range(2):
            ch = i * 2 + b
            base = row0 + ch * _K
            wait_gathers(b)
            out_cp = pltpu.make_async_copy(
                rows_v[b], out_hbm.at[pl.ds(base * _IW, _CHUNK)], osem[b])
            out_cp.start()
            nxt = ch + 2

            @pl.when(nxt < _N_CHUNK)
            def _stage_idx():
                pltpu.sync_copy(
                    idx_hbm.at[pl.ds(row0 + nxt * _K, _K)], idx_v[b])

            out_cp.wait()

            @pl.when(nxt < _N_CHUNK)
            def _fire_next():
                fire_gathers(b)

        return carry

    lax.fori_loop(0, _N_CHUNK // 2, pair_body, 0)


def kernel(input_ids, embed_tokens_weight):
    idx2d = input_ids.reshape(_TOT // _IW, _IW)
    out = _lookup(embed_tokens_weight, idx2d)
    return out.reshape(_BATCH, _HIST, _D)
